# Initial kernel scaffold; baseline (speedup 1.0000x reference)
import jax
import jax.numpy as jnp
from jax.experimental import pallas as pl


def kernel(x, router_logits, W, lora_A, lora_B):
    raise NotImplementedError("write your pallas kernel here")


# speedup vs baseline: 11.5659x; 11.5659x over previous
import functools

import jax
import jax.numpy as jnp
from jax.experimental import pallas as pl

_T = 16384
_D = 1024
_DO = 1024
_E = 64
_K = 8
_SCALE = 2.0
_TB = 512


def _fused_kernel(x_ref, lg_ref, w_ref, a_ref, b_ref, o_ref):
    x = x_ref[:]
    l = lg_ref[:]

    m = jnp.max(l, axis=1, keepdims=True)
    e = jnp.exp(l - m)
    s = e / jnp.sum(e, axis=1, keepdims=True)

    eidx = jax.lax.broadcasted_iota(jnp.int32, l.shape, 1)
    work = l
    mask = jnp.zeros(l.shape, dtype=jnp.bool_)
    for _ in range(_K):
        row_max = jnp.max(work, axis=1, keepdims=True)
        is_max = work == row_max
        cand = jnp.where(is_max, eidx, _E)
        first = jnp.min(cand, axis=1, keepdims=True)
        chosen = eidx == first
        mask = jnp.logical_or(mask, chosen)
        work = jnp.where(chosen, -jnp.inf, work)

    g = jnp.where(mask, s, 0.0)

    hx = jax.lax.dot_general(
        x, a_ref[:], (((1,), (1,)), ((), ())),
        preferred_element_type=jnp.float32)
    ge = g * hx

    lora = jax.lax.dot_general(
        ge, b_ref[:], (((1,), (0,)), ((), ())),
        preferred_element_type=jnp.float32)

    base = jax.lax.dot_general(
        x, w_ref[:], (((1,), (1,)), ((), ())),
        preferred_element_type=jnp.float32)

    o_ref[:] = base + _SCALE * lora


@jax.jit
def kernel(x, router_logits, W, lora_A, lora_B):
    grid = (_T // _TB,)
    return pl.pallas_call(
        _fused_kernel,
        grid=grid,
        in_specs=[
            pl.BlockSpec((_TB, _D), lambda i: (i, 0)),
            pl.BlockSpec((_TB, _E), lambda i: (i, 0)),
            pl.BlockSpec((_DO, _D), lambda i: (0, 0)),
            pl.BlockSpec((_E, _D), lambda i: (0, 0)),
            pl.BlockSpec((_E, _DO), lambda i: (0, 0)),
        ],
        out_specs=pl.BlockSpec((_TB, _DO), lambda i: (i, 0)),
        out_shape=jax.ShapeDtypeStruct((_T, _DO), jnp.float32),
    )(x, router_logits, W, lora_A, lora_B)


# speedup vs baseline: 11.7034x; 1.0119x over previous
import functools

import jax
import jax.numpy as jnp
from jax.experimental import pallas as pl

_T = 16384
_D = 1024
_DO = 1024
_E = 64
_K = 8
_SCALE = 2.0
_TB = 512


def _fused_kernel(x_ref, lg_ref, w_ref, a_ref, b_ref, o_ref):
    x = x_ref[:]
    l = lg_ref[:]

    m = jnp.max(l, axis=1, keepdims=True)
    e = jnp.exp(l - m)
    s = e / jnp.sum(e, axis=1, keepdims=True)

    eidx = jax.lax.broadcasted_iota(jnp.int32, l.shape, 1)
    work = l
    mask = jnp.zeros(l.shape, dtype=jnp.bool_)
    for _ in range(_K):
        row_max = jnp.max(work, axis=1, keepdims=True)
        is_max = work == row_max
        cand = jnp.where(is_max, eidx, _E)
        first = jnp.min(cand, axis=1, keepdims=True)
        chosen = eidx == first
        mask = jnp.logical_or(mask, chosen)
        work = jnp.where(chosen, -jnp.inf, work)

    g = jnp.where(mask, s, 0.0)

    xb = x.astype(jnp.bfloat16)

    hx = jax.lax.dot_general(
        xb, a_ref[:].astype(jnp.bfloat16), (((1,), (1,)), ((), ())),
        preferred_element_type=jnp.float32)
    ge = (g * hx).astype(jnp.bfloat16)

    lora = jax.lax.dot_general(
        ge, b_ref[:].astype(jnp.bfloat16), (((1,), (0,)), ((), ())),
        preferred_element_type=jnp.float32)

    base = jax.lax.dot_general(
        xb, w_ref[:].astype(jnp.bfloat16), (((1,), (1,)), ((), ())),
        preferred_element_type=jnp.float32)

    o_ref[:] = base + _SCALE * lora


@jax.jit
def kernel(x, router_logits, W, lora_A, lora_B):
    grid = (_T // _TB,)
    return pl.pallas_call(
        _fused_kernel,
        grid=grid,
        in_specs=[
            pl.BlockSpec((_TB, _D), lambda i: (i, 0)),
            pl.BlockSpec((_TB, _E), lambda i: (i, 0)),
            pl.BlockSpec((_DO, _D), lambda i: (0, 0)),
            pl.BlockSpec((_E, _D), lambda i: (0, 0)),
            pl.BlockSpec((_E, _DO), lambda i: (0, 0)),
        ],
        out_specs=pl.BlockSpec((_TB, _DO), lambda i: (i, 0)),
        out_shape=jax.ShapeDtypeStruct((_T, _DO), jnp.float32),
    )(x, router_logits, W, lora_A, lora_B)


# speedup vs baseline: 13.9168x; 1.1891x over previous
import functools

import jax
import jax.numpy as jnp
from jax.experimental import pallas as pl

_T = 16384
_D = 1024
_DO = 1024
_E = 64
_K = 8
_SCALE = 2.0
_TB = 512


def _fused_kernel(x_ref, lg_ref, w_ref, a_ref, b_ref, o_ref):
    x = x_ref[:]
    l = lg_ref[:]

    m = jnp.max(l, axis=1, keepdims=True)
    e = jnp.exp(l - m)
    s = e / jnp.sum(e, axis=1, keepdims=True)

    eidx = jax.lax.broadcasted_iota(jnp.int32, l.shape, 1).astype(jnp.float32)
    work = l
    mask = jnp.zeros(l.shape, dtype=jnp.bool_)
    for _ in range(_K):
        row_max = jnp.max(work, axis=1, keepdims=True)
        is_max = work == row_max
        cand = jnp.where(is_max, eidx, float(_E))
        first = jnp.min(cand, axis=1, keepdims=True)
        chosen = eidx == first
        mask = jnp.logical_or(mask, chosen)
        work = jnp.where(chosen, -jnp.inf, work)

    g = jnp.where(mask, s, 0.0)

    xb = x.astype(jnp.bfloat16)

    hx = jax.lax.dot_general(
        xb, a_ref[:].astype(jnp.bfloat16), (((1,), (1,)), ((), ())),
        preferred_element_type=jnp.float32)
    ge = (g * hx).astype(jnp.bfloat16)

    lora = jax.lax.dot_general(
        ge, b_ref[:].astype(jnp.bfloat16), (((1,), (0,)), ((), ())),
        preferred_element_type=jnp.float32)

    base = jax.lax.dot_general(
        xb, w_ref[:].astype(jnp.bfloat16), (((1,), (1,)), ((), ())),
        preferred_element_type=jnp.float32)

    o_ref[:] = base + _SCALE * lora


@jax.jit
def kernel(x, router_logits, W, lora_A, lora_B):
    grid = (_T // _TB,)
    return pl.pallas_call(
        _fused_kernel,
        grid=grid,
        in_specs=[
            pl.BlockSpec((_TB, _D), lambda i: (i, 0)),
            pl.BlockSpec((_TB, _E), lambda i: (i, 0)),
            pl.BlockSpec((_DO, _D), lambda i: (0, 0)),
            pl.BlockSpec((_E, _D), lambda i: (0, 0)),
            pl.BlockSpec((_E, _DO), lambda i: (0, 0)),
        ],
        out_specs=pl.BlockSpec((_TB, _DO), lambda i: (i, 0)),
        out_shape=jax.ShapeDtypeStruct((_T, _DO), jnp.float32),
    )(x, router_logits, W, lora_A, lora_B)


# speedup vs baseline: 16.9765x; 1.2199x over previous
import functools

import jax
import jax.numpy as jnp
from jax.experimental import pallas as pl

_T = 16384
_D = 1024
_DO = 1024
_E = 64
_K = 8
_SCALE = 2.0
_TB = 512


def _fused_kernel(x_ref, lg_ref, w_ref, a_ref, b_ref, o_ref):
    x = x_ref[:]
    l = lg_ref[:]

    m = jnp.max(l, axis=1, keepdims=True)
    e = jnp.exp(l - m)
    s = e / jnp.sum(e, axis=1, keepdims=True)

    work = l
    for _ in range(_K - 1):
        row_max = jnp.max(work, axis=1, keepdims=True)
        work = jnp.where(work == row_max, -jnp.inf, work)
    t_k = jnp.max(work, axis=1, keepdims=True)

    g = jnp.where(l >= t_k, s, 0.0)

    xb = x.astype(jnp.bfloat16)

    hx = jax.lax.dot_general(
        xb, a_ref[:].astype(jnp.bfloat16), (((1,), (1,)), ((), ())),
        preferred_element_type=jnp.float32)
    ge = (g * hx).astype(jnp.bfloat16)

    lora = jax.lax.dot_general(
        ge, b_ref[:].astype(jnp.bfloat16), (((1,), (0,)), ((), ())),
        preferred_element_type=jnp.float32)

    base = jax.lax.dot_general(
        xb, w_ref[:].astype(jnp.bfloat16), (((1,), (1,)), ((), ())),
        preferred_element_type=jnp.float32)

    o_ref[:] = base + _SCALE * lora


@jax.jit
def kernel(x, router_logits, W, lora_A, lora_B):
    grid = (_T // _TB,)
    return pl.pallas_call(
        _fused_kernel,
        grid=grid,
        in_specs=[
            pl.BlockSpec((_TB, _D), lambda i: (i, 0)),
            pl.BlockSpec((_TB, _E), lambda i: (i, 0)),
            pl.BlockSpec((_DO, _D), lambda i: (0, 0)),
            pl.BlockSpec((_E, _D), lambda i: (0, 0)),
            pl.BlockSpec((_E, _DO), lambda i: (0, 0)),
        ],
        out_specs=pl.BlockSpec((_TB, _DO), lambda i: (i, 0)),
        out_shape=jax.ShapeDtypeStruct((_T, _DO), jnp.float32),
    )(x, router_logits, W, lora_A, lora_B)


# speedup vs baseline: 18.6687x; 1.0997x over previous
import functools

import jax
import jax.numpy as jnp
from jax.experimental import pallas as pl

_T = 16384
_D = 1024
_DO = 1024
_E = 64
_K = 8
_SCALE = 2.0
_TB = 1024


def _fused_kernel(x_ref, lg_ref, w_ref, a_ref, b_ref, o_ref):
    x = x_ref[:]
    l = lg_ref[:]

    xb = x.astype(jnp.bfloat16)

    hx = jax.lax.dot_general(
        xb, a_ref[:].astype(jnp.bfloat16), (((1,), (1,)), ((), ())),
        preferred_element_type=jnp.float32)

    m = jnp.max(l, axis=1, keepdims=True)
    e = jnp.exp(l - m)
    s = e / jnp.sum(e, axis=1, keepdims=True)

    work = l
    for _ in range(_K - 1):
        row_max = jnp.max(work, axis=1, keepdims=True)
        work = jnp.where(work == row_max, -jnp.inf, work)
    t_k = jnp.max(work, axis=1, keepdims=True)

    g = jnp.where(l >= t_k, s, 0.0)
    ge = (g * hx).astype(jnp.bfloat16)

    lora = jax.lax.dot_general(
        ge, b_ref[:].astype(jnp.bfloat16), (((1,), (0,)), ((), ())),
        preferred_element_type=jnp.float32)

    base = jax.lax.dot_general(
        xb, w_ref[:].astype(jnp.bfloat16), (((1,), (1,)), ((), ())),
        preferred_element_type=jnp.float32)

    o_ref[:] = base + _SCALE * lora


@jax.jit
def kernel(x, router_logits, W, lora_A, lora_B):
    grid = (_T // _TB,)
    return pl.pallas_call(
        _fused_kernel,
        grid=grid,
        in_specs=[
            pl.BlockSpec((_TB, _D), lambda i: (i, 0)),
            pl.BlockSpec((_TB, _E), lambda i: (i, 0)),
            pl.BlockSpec((_DO, _D), lambda i: (0, 0)),
            pl.BlockSpec((_E, _D), lambda i: (0, 0)),
            pl.BlockSpec((_E, _DO), lambda i: (0, 0)),
        ],
        out_specs=pl.BlockSpec((_TB, _DO), lambda i: (i, 0)),
        out_shape=jax.ShapeDtypeStruct((_T, _DO), jnp.float32),
    )(x, router_logits, W, lora_A, lora_B)


# speedup vs baseline: 18.6855x; 1.0009x over previous
import functools

import jax
import jax.numpy as jnp
from jax.experimental import pallas as pl

_T = 16384
_D = 1024
_DO = 1024
_E = 64
_K = 8
_SCALE = 2.0
_TB = 2048


def _fused_kernel(x_ref, lg_ref, w_ref, a_ref, b_ref, o_ref):
    x = x_ref[:]
    l = lg_ref[:]

    xb = x.astype(jnp.bfloat16)

    hx = jax.lax.dot_general(
        xb, a_ref[:].astype(jnp.bfloat16), (((1,), (1,)), ((), ())),
        preferred_element_type=jnp.float32)

    m = jnp.max(l, axis=1, keepdims=True)
    e = jnp.exp(l - m)
    s = e / jnp.sum(e, axis=1, keepdims=True)

    work = l
    for _ in range(_K - 1):
        row_max = jnp.max(work, axis=1, keepdims=True)
        work = jnp.where(work == row_max, -jnp.inf, work)
    t_k = jnp.max(work, axis=1, keepdims=True)

    g = jnp.where(l >= t_k, s, 0.0)
    ge = (g * hx).astype(jnp.bfloat16)

    lora = jax.lax.dot_general(
        ge, b_ref[:].astype(jnp.bfloat16), (((1,), (0,)), ((), ())),
        preferred_element_type=jnp.float32)

    base = jax.lax.dot_general(
        xb, w_ref[:].astype(jnp.bfloat16), (((1,), (1,)), ((), ())),
        preferred_element_type=jnp.float32)

    o_ref[:] = base + _SCALE * lora


@jax.jit
def kernel(x, router_logits, W, lora_A, lora_B):
    grid = (_T // _TB,)
    return pl.pallas_call(
        _fused_kernel,
        grid=grid,
        in_specs=[
            pl.BlockSpec((_TB, _D), lambda i: (i, 0)),
            pl.BlockSpec((_TB, _E), lambda i: (i, 0)),
            pl.BlockSpec((_DO, _D), lambda i: (0, 0)),
            pl.BlockSpec((_E, _D), lambda i: (0, 0)),
            pl.BlockSpec((_E, _DO), lambda i: (0, 0)),
        ],
        out_specs=pl.BlockSpec((_TB, _DO), lambda i: (i, 0)),
        out_shape=jax.ShapeDtypeStruct((_T, _DO), jnp.float32),
    )(x, router_logits, W, lora_A, lora_B)


# speedup vs baseline: 22.2624x; 1.1914x over previous
import jax
import jax.numpy as jnp
from jax.experimental import pallas as pl

_T = 16384
_D = 1024
_DO = 1024
_E = 64
_K = 8
_SCALE = 2.0
_TB = 1024


def _fused_kernel(x_ref, lt_ref, w_ref, a_ref, b_ref, o_ref):
    x = x_ref[:]
    lt = lt_ref[:]

    xb = x.astype(jnp.bfloat16)

    hx_t = jax.lax.dot_general(
        a_ref[:], xb, (((1,), (1,)), ((), ())),
        preferred_element_type=jnp.float32)

    m = jnp.max(lt, axis=0, keepdims=True)
    e = jnp.exp(lt - m)
    s = e / jnp.sum(e, axis=0, keepdims=True)

    work = lt
    for _ in range(_K - 1):
        row_max = jnp.max(work, axis=0, keepdims=True)
        work = jnp.where(work == row_max, -jnp.inf, work)
    t_k = jnp.max(work, axis=0, keepdims=True)

    g_t = jnp.where(lt >= t_k, s, 0.0)
    ge_t = (g_t * hx_t).astype(jnp.bfloat16)

    lora = jax.lax.dot_general(
        ge_t, b_ref[:], (((0,), (0,)), ((), ())),
        preferred_element_type=jnp.float32)

    base = jax.lax.dot_general(
        xb, w_ref[:], (((1,), (1,)), ((), ())),
        preferred_element_type=jnp.float32)

    o_ref[:] = base + _SCALE * lora


@jax.jit
def kernel(x, router_logits, W, lora_A, lora_B):
    lt = router_logits.T
    wb = W.astype(jnp.bfloat16)
    ab = lora_A.astype(jnp.bfloat16)
    bb = lora_B.astype(jnp.bfloat16)
    grid = (_T // _TB,)
    return pl.pallas_call(
        _fused_kernel,
        grid=grid,
        in_specs=[
            pl.BlockSpec((_TB, _D), lambda i: (i, 0)),
            pl.BlockSpec((_E, _TB), lambda i: (0, i)),
            pl.BlockSpec((_DO, _D), lambda i: (0, 0)),
            pl.BlockSpec((_E, _D), lambda i: (0, 0)),
            pl.BlockSpec((_E, _DO), lambda i: (0, 0)),
        ],
        out_specs=pl.BlockSpec((_TB, _DO), lambda i: (i, 0)),
        out_shape=jax.ShapeDtypeStruct((_T, _DO), jnp.float32),
    )(x, lt, wb, ab, bb)


# speedup vs baseline: 22.3568x; 1.0042x over previous
import jax
import jax.numpy as jnp
from jax.experimental import pallas as pl
from jax.experimental.pallas import tpu as pltpu

_T = 16384
_D = 1024
_DO = 1024
_E = 64
_K = 8
_SCALE = 2.0
_TB = 1024


def _fused_kernel(x_ref, lt_ref, w_ref, a_ref, b_ref, o_ref):
    x = x_ref[:]
    lt = lt_ref[:]

    xb = x.astype(jnp.bfloat16)

    hx_t = jax.lax.dot_general(
        a_ref[:], xb, (((1,), (1,)), ((), ())),
        preferred_element_type=jnp.float32)

    m = jnp.max(lt, axis=0, keepdims=True)
    e = jnp.exp(lt - m)
    s = e / jnp.sum(e, axis=0, keepdims=True)

    work = lt
    for _ in range(_K - 1):
        row_max = jnp.max(work, axis=0, keepdims=True)
        work = jnp.where(work == row_max, -jnp.inf, work)
    t_k = jnp.max(work, axis=0, keepdims=True)

    g_t = jnp.where(lt >= t_k, s, 0.0)
    ge_t = (g_t * hx_t).astype(jnp.bfloat16)

    lora = jax.lax.dot_general(
        ge_t, b_ref[:], (((0,), (0,)), ((), ())),
        preferred_element_type=jnp.float32)

    base = jax.lax.dot_general(
        xb, w_ref[:], (((1,), (1,)), ((), ())),
        preferred_element_type=jnp.float32)

    o_ref[:] = base + _SCALE * lora


@jax.jit
def kernel(x, router_logits, W, lora_A, lora_B):
    lt = router_logits.T
    wb = W.astype(jnp.bfloat16)
    ab = lora_A.astype(jnp.bfloat16)
    bb = lora_B.astype(jnp.bfloat16)
    grid = (_T // _TB,)
    return pl.pallas_call(
        _fused_kernel,
        grid=grid,
        in_specs=[
            pl.BlockSpec((_TB, _D), lambda i: (i, 0)),
            pl.BlockSpec((_E, _TB), lambda i: (0, i)),
            pl.BlockSpec((_DO, _D), lambda i: (0, 0)),
            pl.BlockSpec((_E, _D), lambda i: (0, 0)),
            pl.BlockSpec((_E, _DO), lambda i: (0, 0)),
        ],
        out_specs=pl.BlockSpec((_TB, _DO), lambda i: (i, 0)),
        out_shape=jax.ShapeDtypeStruct((_T, _DO), jnp.float32),
        compiler_params=pltpu.CompilerParams(
            dimension_semantics=("parallel",)),
    )(x, lt, wb, ab, bb)


# speedup vs baseline: 22.6260x; 1.0120x over previous
import jax
import jax.numpy as jnp
from jax.experimental import pallas as pl
from jax.experimental.pallas import tpu as pltpu

_T = 16384
_D = 1024
_DO = 1024
_E = 64
_K = 8
_SCALE = 2.0
_TB = 2048


def _fused_kernel(x_ref, lt_ref, w_ref, a_ref, b_ref, o_ref):
    x = x_ref[:]
    lt = lt_ref[:]

    xb = x.astype(jnp.bfloat16)

    hx_t = jax.lax.dot_general(
        a_ref[:], xb, (((1,), (1,)), ((), ())),
        preferred_element_type=jnp.float32)

    m = jnp.max(lt, axis=0, keepdims=True)
    e = jnp.exp(lt - m)
    s = e / jnp.sum(e, axis=0, keepdims=True)

    work = lt
    for _ in range(_K - 1):
        row_max = jnp.max(work, axis=0, keepdims=True)
        work = jnp.where(work == row_max, -jnp.inf, work)
    t_k = jnp.max(work, axis=0, keepdims=True)

    g_t = jnp.where(lt >= t_k, s, 0.0)
    ge_t = (g_t * hx_t).astype(jnp.bfloat16)

    lora = jax.lax.dot_general(
        ge_t, b_ref[:], (((0,), (0,)), ((), ())),
        preferred_element_type=jnp.float32)

    base = jax.lax.dot_general(
        xb, w_ref[:], (((1,), (1,)), ((), ())),
        preferred_element_type=jnp.float32)

    o_ref[:] = base + _SCALE * lora


@jax.jit
def kernel(x, router_logits, W, lora_A, lora_B):
    lt = router_logits.T
    wb = W.astype(jnp.bfloat16)
    ab = lora_A.astype(jnp.bfloat16)
    bb = lora_B.astype(jnp.bfloat16)
    grid = (_T // _TB,)
    return pl.pallas_call(
        _fused_kernel,
        grid=grid,
        in_specs=[
            pl.BlockSpec((_TB, _D), lambda i: (i, 0)),
            pl.BlockSpec((_E, _TB), lambda i: (0, i)),
            pl.BlockSpec((_DO, _D), lambda i: (0, 0)),
            pl.BlockSpec((_E, _D), lambda i: (0, 0)),
            pl.BlockSpec((_E, _DO), lambda i: (0, 0)),
        ],
        out_specs=pl.BlockSpec((_TB, _DO), lambda i: (i, 0)),
        out_shape=jax.ShapeDtypeStruct((_T, _DO), jnp.float32),
        compiler_params=pltpu.CompilerParams(
            dimension_semantics=("parallel",)),
    )(x, lt, wb, ab, bb)


# speedup vs baseline: 25.1047x; 1.1096x over previous
import jax
import jax.numpy as jnp
from jax.experimental import pallas as pl
from jax.experimental.pallas import tpu as pltpu

_T = 16384
_D = 1024
_DO = 1024
_E = 64
_K = 8
_SCALE = 2.0
_TB = 2048


def _fused_kernel(x_ref, lt_ref, w_ref, a_ref, b_ref, o_ref):
    x = x_ref[:]
    lt = lt_ref[:]

    hx_t = jax.lax.dot_general(
        a_ref[:], x, (((1,), (1,)), ((), ())),
        preferred_element_type=jnp.float32,
        precision=jax.lax.Precision.DEFAULT)

    m = jnp.max(lt, axis=0, keepdims=True)
    e = jnp.exp(lt - m)
    s = e / jnp.sum(e, axis=0, keepdims=True)

    work = lt
    for _ in range(_K - 1):
        row_max = jnp.max(work, axis=0, keepdims=True)
        work = jnp.where(work == row_max, -jnp.inf, work)
    t_k = jnp.max(work, axis=0, keepdims=True)

    g_t = jnp.where(lt >= t_k, s, 0.0)
    ge_t = g_t * hx_t

    lora = jax.lax.dot_general(
        ge_t, b_ref[:], (((0,), (0,)), ((), ())),
        preferred_element_type=jnp.float32,
        precision=jax.lax.Precision.DEFAULT)

    base = jax.lax.dot_general(
        x, w_ref[:], (((1,), (1,)), ((), ())),
        preferred_element_type=jnp.float32,
        precision=jax.lax.Precision.DEFAULT)

    o_ref[:] = base + _SCALE * lora


@jax.jit
def kernel(x, router_logits, W, lora_A, lora_B):
    lt = router_logits.T
    grid = (_T // _TB,)
    return pl.pallas_call(
        _fused_kernel,
        grid=grid,
        in_specs=[
            pl.BlockSpec((_TB, _D), lambda i: (i, 0)),
            pl.BlockSpec((_E, _TB), lambda i: (0, i)),
            pl.BlockSpec((_DO, _D), lambda i: (0, 0)),
            pl.BlockSpec((_E, _D), lambda i: (0, 0)),
            pl.BlockSpec((_E, _DO), lambda i: (0, 0)),
        ],
        out_specs=pl.BlockSpec((_TB, _DO), lambda i: (i, 0)),
        out_shape=jax.ShapeDtypeStruct((_T, _DO), jnp.float32),
        compiler_params=pltpu.CompilerParams(
            dimension_semantics=("parallel",)),
    )(x, lt, W, lora_A, lora_B)
